# Initial kernel scaffold; baseline (speedup 1.0000x reference)
import jax
import jax.numpy as jnp
from jax.experimental import pallas as pl


def kernel(x, params):
    raise NotImplementedError("write your pallas kernel here")


# speedup vs baseline: 9.5409x; 9.5409x over previous
import numpy as np
import jax
import jax.numpy as jnp
from jax import lax
from jax.experimental import pallas as pl

T, B, NPOKE, NFEAT = 128, 64, 18, 31
DIM, AE = 128, 64
ROWS = T * B * NPOKE

VOC_POKE, VOC_ABIL, VOC_ITEM, VOC_MOVE = 1300, 300, 500, 900
N_ITEM_EFF, N_STATUS, N_GENDER, N_FORMES, N_TYPES = 18, 7, 3, 50, 19


def _sqrt_oh(n):
    idx = np.floor(np.sqrt(np.arange(n))).astype(np.int64)
    return np.eye(int(idx.max()) + 1, dtype=np.float32)[idx]


def _pow_oh(n, p):
    idx = np.floor(np.arange(n) ** p).astype(np.int64)
    return np.eye(int(idx.max()) + 1, dtype=np.float32)[idx]


_HP_OH = _sqrt_oh(768)[:, 1:]
_STAT_OH = _pow_oh(512, 1.0 / 3.0)[:, 1:]
_STATUS_OH = np.eye(N_STATUS + 1, dtype=np.float32)[:, 1:]
_SLEEP_OH = np.eye(4, dtype=np.float32)[:, 1:]
_TOXIC_OH = _sqrt_oh(16)[:, 1:]
_ITEMEFF_OH = np.eye(N_ITEM_EFF + 1, dtype=np.float32)[:, 1:]

def _seg(sizes):
    offs, o = {}, 0
    for name, n, pad in sizes:
        offs[name] = o
        o += -(-n // pad) * pad
    return offs, o

_SEGS = [
    ("pokedex", VOC_POKE, 8),
    ("ability", VOC_ABIL, 8),
    ("item", VOC_ITEM, 8),
    ("lastmove", VOC_MOVE, 8),
    ("move", VOC_MOVE, 8),
    ("hp2", 2 * 768, 8),
    ("stat5", 5 * 512, 8),
    ("small", 232, 8),
]
_OFF, _TOT = _seg(_SEGS)
_TOTP = -(-_TOT // 8) * 8

_SM = {}
_o = 0
for _nm, _n in [("status", 8), ("sleep", 4), ("toxic", 16), ("itemeff", 19),
                ("forme", N_FORMES), ("active", 3), ("fainted", 3),
                ("gender", N_GENDER + 1), ("level", 102),
                ("teratype", N_TYPES + 1), ("tera", 2), ("statrow", 1)]:
    _SM[_nm] = _o
    _o += _n
assert _o == 232


def _fusion_body(po_t, po_w, po_b, ab_t, ab_w, ab_b, it_t, it_w, it_b,
                 mv_t, mv_w, mv_b, lm_w, lm_b, st_w, st_b, su_w, su_b,
                 forme_t, active_t, fainted_t, gender_t, level_t,
                 teratype_t, tera_t,
                 hp_oh, stat_oh, status_oh, sleep_oh, toxic_oh, itemeff_oh,
                 ft_ref):
    f32 = jnp.float32
    dot = lambda a, b: jnp.dot(a[...], b[...], preferred_element_type=f32)
    ft_ref[...] = jnp.zeros((_TOTP, DIM), f32)
    ft_ref[_OFF["pokedex"]:_OFF["pokedex"] + VOC_POKE, :] = dot(po_t, po_w) + po_b[...]
    ft_ref[_OFF["ability"]:_OFF["ability"] + VOC_ABIL, :] = dot(ab_t, ab_w) + ab_b[...]
    ft_ref[_OFF["item"]:_OFF["item"] + VOC_ITEM, :] = dot(it_t, it_w[0:AE, :]) + it_b[...]
    ft_ref[_OFF["lastmove"]:_OFF["lastmove"] + VOC_MOVE, :] = dot(mv_t, lm_w) + lm_b[...]
    ft_ref[_OFF["move"]:_OFF["move"] + VOC_MOVE, :] = dot(mv_t, mv_w) + mv_b[...]
    ft_ref[_OFF["hp2"]:_OFF["hp2"] + 768, :] = dot(hp_oh, st_w[0:27, :]) + st_b[...]
    ft_ref[_OFF["hp2"] + 768:_OFF["hp2"] + 1536, :] = dot(hp_oh, st_w[27:54, :])
    for k in range(5):
        ft_ref[_OFF["stat5"] + 512 * k:_OFF["stat5"] + 512 * (k + 1), :] = (
            dot(stat_oh, st_w[55 + 7 * k:62 + 7 * k, :]))
    s = _OFF["small"]
    ft_ref[s + _SM["status"]:s + _SM["status"] + 8, :] = dot(status_oh, su_w[0:7, :]) + su_b[...]
    ft_ref[s + _SM["sleep"]:s + _SM["sleep"] + 4, :] = dot(sleep_oh, su_w[7:10, :])
    ft_ref[s + _SM["toxic"]:s + _SM["toxic"] + 16, :] = dot(toxic_oh, su_w[10:13, :])
    ft_ref[s + _SM["itemeff"]:s + _SM["itemeff"] + 19, :] = dot(itemeff_oh, it_w[AE:AE + 18, :])
    ft_ref[s + _SM["forme"]:s + _SM["forme"] + N_FORMES, :] = forme_t[...]
    ft_ref[s + _SM["active"]:s + _SM["active"] + 3, :] = active_t[...]
    ft_ref[s + _SM["fainted"]:s + _SM["fainted"] + 3, :] = fainted_t[...]
    ft_ref[s + _SM["gender"]:s + _SM["gender"] + N_GENDER + 1, :] = gender_t[...]
    ft_ref[s + _SM["level"]:s + _SM["level"] + 102, :] = level_t[...]
    ft_ref[s + _SM["teratype"]:s + _SM["teratype"] + N_TYPES + 1, :] = teratype_t[...]
    ft_ref[s + _SM["tera"]:s + _SM["tera"] + 2, :] = tera_t[...]
    ft_ref[s + _SM["statrow"]:s + _SM["statrow"] + 1, :] = st_w[54:55, :]


def _build_ft(p):
    consts = [jnp.asarray(a) for a in
              (_HP_OH, _STAT_OH, _STATUS_OH, _SLEEP_OH, _TOXIC_OH, _ITEMEFF_OH)]
    args = [p["pokedex_tab"], p["pokedex_W"], p["pokedex_b"].reshape(1, DIM),
            p["ability_tab"], p["ability_W"], p["ability_b"].reshape(1, DIM),
            p["item_tab"], p["item_W"], p["item_b"].reshape(1, DIM),
            p["move_tab"], p["move_W"], p["move_b"].reshape(1, DIM),
            p["lastmove_W"], p["lastmove_b"].reshape(1, DIM),
            p["stat_W"], p["stat_b"].reshape(1, DIM),
            p["status_W"], p["status_b"].reshape(1, DIM),
            p["forme_tab"], p["active_tab"], p["fainted_tab"],
            p["gender_tab"], p["level_tab"], p["teratype_tab"], p["tera_tab"],
            ] + consts
    return pl.pallas_call(
        _fusion_body,
        out_shape=jax.ShapeDtypeStruct((_TOTP, DIM), jnp.float32),
    )(*args)


_R = 512
_NB = ROWS // _R


def _oh(idx, V):
    i2 = lax.broadcasted_iota(jnp.int32, (_R, V), 1)
    return (idx == i2).astype(jnp.bfloat16)


def _main_body(x_ref, ft_ref, w1_ref, b1_ref, g_ref, lb_ref, w2_ref, b2_ref,
               out_ref, mv_ref):
    f32 = jnp.float32
    xb = x_ref[...]
    longs = (xb + 1.0).astype(jnp.int32)
    col = lambda c: longs[:, c:c + 1]

    bf = jnp.bfloat16
    dotf = lambda a, b: jnp.dot(a, b, preferred_element_type=f32)

    pemb = dotf(_oh(col(0), VOC_POKE), ft_ref[_OFF["pokedex"]:_OFF["pokedex"] + VOC_POKE, :])
    pemb += dotf(_oh(col(15), VOC_ABIL), ft_ref[_OFF["ability"]:_OFF["ability"] + VOC_ABIL, :])
    pemb += dotf(_oh(col(17), VOC_ITEM), ft_ref[_OFF["item"]:_OFF["item"] + VOC_ITEM, :])
    pemb += dotf(_oh(col(24), VOC_MOVE), ft_ref[_OFF["lastmove"]:_OFF["lastmove"] + VOC_MOVE, :])

    hp2 = jnp.concatenate([_oh(col(3), 768), _oh(col(4), 768)], axis=1)
    pemb += dotf(hp2, ft_ref[_OFF["hp2"]:_OFF["hp2"] + 1536, :])

    st = jnp.concatenate([_oh(col(6 + k), 512) for k in range(5)], axis=1)
    pemb += dotf(st, ft_ref[_OFF["stat5"]:_OFF["stat5"] + 2560, :])

    i2 = lax.broadcasted_iota(jnp.int32, (_R, 232), 1)
    mh = jnp.zeros((_R, 232), f32)
    for nm, c in (("status", 21), ("sleep", 22), ("toxic", 23), ("itemeff", 19),
                  ("forme", 1), ("active", 12), ("fainted", 11), ("gender", 14),
                  ("level", 13), ("teratype", 30)):
        mh += (i2 == col(c) + _SM[nm]).astype(f32)
    tera_idx = (col(29) > 0).astype(jnp.int32)
    mh += (i2 == tera_idx + _SM["tera"]).astype(f32)
    ratio = xb[:, 5:6]
    mh += (i2 == _SM["statrow"]).astype(f32) * ratio
    pemb += dotf(mh.astype(bf), ft_ref[_OFF["small"]:_OFF["small"] + 232, :])

    mvtab = ft_ref[_OFF["move"]:_OFF["move"] + VOC_MOVE, :]
    cnt = jnp.zeros((_R, 1), f32)
    msum = jnp.zeros((_R, DIM), f32)
    for m in range(4):
        me = dotf(_oh(col(25 + m), VOC_MOVE), mvtab)
        mv_ref[:, m, :] = me
        msum += me
        cnt += (col(25 + m) > 0).astype(f32)
    pemb += msum / jnp.maximum(cnt, 1.0)

    h = dotf(pemb, w1_ref[...]) + b1_ref[...]
    h = jnp.maximum(h, 0.0)
    mu = jnp.mean(h, axis=-1, keepdims=True)
    var = jnp.mean((h - mu) ** 2, axis=-1, keepdims=True)
    h = (h - mu) * lax.rsqrt(var + 1e-5) * g_ref[...] + lb_ref[...]
    out_ref[...] = dotf(h, w2_ref[...]) + b2_ref[...]


def kernel(x, params):
    p = params
    ft = _build_ft(p).astype(jnp.bfloat16)
    x2 = x.reshape(ROWS, NFEAT)

    row_spec = pl.BlockSpec((_R, NFEAT), lambda i: (i, 0))
    full = lambda shape: pl.BlockSpec(shape, lambda i: tuple(0 for _ in shape))
    out, mv = pl.pallas_call(
        _main_body,
        grid=(_NB,),
        in_specs=[row_spec, full((_TOTP, DIM)), full((DIM, DIM)),
                  full((1, DIM)), full((1, DIM)), full((1, DIM)),
                  full((DIM, DIM)), full((1, DIM))],
        out_specs=[pl.BlockSpec((_R, DIM), lambda i: (i, 0)),
                   pl.BlockSpec((_R, 4, DIM), lambda i: (i, 0, 0))],
        out_shape=[jax.ShapeDtypeStruct((ROWS, DIM), jnp.float32),
                   jax.ShapeDtypeStruct((ROWS, 4, DIM), jnp.float32)],
    )(x2, ft, p["enc_W1"], p["enc_b1"].reshape(1, DIM),
      p["ln_g"].reshape(1, DIM), p["ln_b"].reshape(1, DIM),
      p["enc_W2"], p["enc_b2"].reshape(1, DIM))

    out = out.reshape(T, B, NPOKE, DIM)
    moves_emb = mv.reshape(T, B, NPOKE, 4, DIM)

    longs = (x + 1.0).astype(jnp.int32)
    mask = (longs[..., 0] == 0) | (longs[..., 11] == 2)
    priv, pub1, pub2 = jnp.split(out, 3, axis=2)
    pm, m1, m2 = jnp.split(mask, 3, axis=2)
    return ((priv, pub1, pub2), (pm, m1, m2), moves_emb)
